# Initial kernel scaffold; baseline (speedup 1.0000x reference)
import jax
import jax.numpy as jnp
from jax.experimental import pallas as pl


def kernel(x, a, edge_attr, u, W, b, edge_index, batch):
    raise NotImplementedError("write your pallas kernel here")


# speedup vs baseline: 3.8547x; 3.8547x over previous
import functools

import jax
import jax.numpy as jnp
from jax import lax
from jax.experimental import pallas as pl
from jax.experimental.pallas import tpu as pltpu
from jax.experimental.pallas import tpu_sc as plsc


def _make_sc_scatter(E, N, FE, C=80):
    info = plsc.get_sparse_core_info()
    NC, NS = info.num_cores, info.num_subcores
    NW = NC * NS
    n_chunks = E // C
    per_tile = n_chunks // NW
    rows = N // NS
    mesh = plsc.VectorSubcoreMesh(core_axis_name="c", subcore_axis_name="s")

    @functools.partial(
        pl.kernel,
        mesh=mesh,
        compiler_params=pltpu.CompilerParams(use_tc_tiling_on_sc=False),
        out_type=[
            jax.ShapeDtypeStruct((NC, N, FE), jnp.float32),
            jax.ShapeDtypeStruct((NC, N, FE), jnp.float32),
        ],
        scratch_types=[
            pltpu.VMEM((C, FE), jnp.float32),
            pltpu.VMEM((C,), jnp.int32),
            pltpu.VMEM((C, FE), jnp.float32),
            pltpu.VMEM_SHARED((N, FE), jnp.float32),
            pltpu.VMEM_SHARED((N, FE), jnp.float32),
        ],
    )
    def sc_scatter(attr_hbm, dest_hbm, zeros_hbm, ones_hbm,
                   psum_hbm, pcnt_hbm,
                   attr_v, idx_v, ones_v, sum_sh, cnt_sh):
        cid = lax.axis_index("c")
        sid = lax.axis_index("s")
        wid = sid * NC + cid
        base = sid * rows
        pltpu.sync_copy(zeros_hbm.at[pl.ds(base, rows)],
                        sum_sh.at[pl.ds(base, rows)])
        pltpu.sync_copy(zeros_hbm.at[pl.ds(base, rows)],
                        cnt_sh.at[pl.ds(base, rows)])
        pltpu.sync_copy(ones_hbm, ones_v)
        plsc.subcore_barrier()

        def body(i, carry):
            chunk = wid * per_tile + i
            off = chunk * C
            pltpu.sync_copy(dest_hbm.at[pl.ds(off, C)], idx_v)
            pltpu.sync_copy(attr_hbm.at[pl.ds(off, C)], attr_v)
            pltpu.sync_copy(attr_v, sum_sh.at[idx_v], add=True)
            pltpu.sync_copy(ones_v, cnt_sh.at[idx_v], add=True)
            return carry

        lax.fori_loop(0, per_tile, body, None)
        plsc.subcore_barrier()
        pltpu.sync_copy(sum_sh.at[pl.ds(base, rows)],
                        psum_hbm.at[cid, pl.ds(base, rows)])
        pltpu.sync_copy(cnt_sh.at[pl.ds(base, rows)],
                        pcnt_hbm.at[cid, pl.ds(base, rows)])

    return sc_scatter


def _dense(x, a, psum, pcnt, u, W, b, batch3, blk=1000):
    N, FX = x.shape
    Bu, FU = u.shape
    FE = psum.shape[-1]
    FOUT = W.shape[1]
    nblk = N // blk

    def body(x_ref, a_ref, ps_ref, pc_ref, u_ref, w_ref, b_ref, bt_ref, o_ref):
        s = ps_ref[0] + ps_ref[1]
        cnt = pc_ref[0][:, 0:1] + pc_ref[1][:, 0:1]
        eagg = s / jnp.maximum(cnt, 1.0)
        bv = bt_ref[0, 0, :]
        oh = (bv[:, None] == lax.broadcasted_iota(jnp.int32, (blk, Bu), 1)
              ).astype(jnp.float32)
        ub = jnp.dot(oh, u_ref[...], preferred_element_type=jnp.float32)
        acc = jnp.dot(x_ref[...], w_ref[0:FX, :],
                      preferred_element_type=jnp.float32)
        acc += jnp.dot(a_ref[...], w_ref[FX:2 * FX, :],
                       preferred_element_type=jnp.float32)
        acc += jnp.dot(eagg, w_ref[2 * FX:2 * FX + FE, :],
                       preferred_element_type=jnp.float32)
        acc += jnp.dot(ub, w_ref[2 * FX + FE:, :],
                       preferred_element_type=jnp.float32)
        o_ref[...] = jnp.maximum(acc + b_ref[...][None, :], 0.0)

    return pl.pallas_call(
        body,
        grid=(nblk,),
        in_specs=[
            pl.BlockSpec((blk, FX), lambda i: (i, 0)),
            pl.BlockSpec((blk, FX), lambda i: (i, 0)),
            pl.BlockSpec((2, blk, FE), lambda i: (0, i, 0)),
            pl.BlockSpec((2, blk, FE), lambda i: (0, i, 0)),
            pl.BlockSpec((Bu, FU), lambda i: (0, 0)),
            pl.BlockSpec(W.shape, lambda i: (0, 0)),
            pl.BlockSpec(b.shape, lambda i: (0,)),
            pl.BlockSpec((1, 1, blk), lambda i: (i, 0, 0)),
        ],
        out_specs=pl.BlockSpec((blk, FOUT), lambda i: (i, 0)),
        out_shape=jax.ShapeDtypeStruct((N, FOUT), jnp.float32),
    )(x, a, psum, pcnt, u, W, b, batch3)


def kernel(x, a, edge_attr, u, W, b, edge_index, batch):
    E, FE = edge_attr.shape
    N = x.shape[0]
    C = 80
    info = plsc.get_sparse_core_info()
    align = info.num_subcores * 8
    n_pad = ((N + align - 1) // align) * align
    dest = edge_index[1]
    zeros = jnp.zeros((n_pad, FE), dtype=jnp.float32)
    ones = jnp.ones((C, FE), dtype=jnp.float32)
    psum, pcnt = _make_sc_scatter(E, n_pad, FE, C)(edge_attr, dest, zeros, ones)
    psum = psum[:, :N]
    pcnt = pcnt[:, :N]
    batch3 = batch.reshape(N // 1000, 1, 1000)
    return _dense(x, a, psum, pcnt, u, W, b, batch3, blk=1000)


# speedup vs baseline: 4.9917x; 1.2950x over previous
import functools

import jax
import jax.numpy as jnp
from jax import lax
from jax.experimental import pallas as pl
from jax.experimental.pallas import tpu as pltpu
from jax.experimental.pallas import tpu_sc as plsc


def _make_sc_scatter(E, N, FE, C=80):
    info = plsc.get_sparse_core_info()
    NC, NS = info.num_cores, info.num_subcores
    NW = NC * NS
    n_chunks = E // C
    per_tile = n_chunks // NW
    rows = N // NS
    mesh = plsc.VectorSubcoreMesh(core_axis_name="c", subcore_axis_name="s")

    @functools.partial(
        pl.kernel,
        mesh=mesh,
        compiler_params=pltpu.CompilerParams(use_tc_tiling_on_sc=False),
        out_type=[
            jax.ShapeDtypeStruct((NC, N, FE), jnp.float32),
            jax.ShapeDtypeStruct((NC, N, FE), jnp.float32),
        ],
        scratch_types=[
            pltpu.VMEM((2, C, FE), jnp.float32),
            pltpu.VMEM((2, C), jnp.int32),
            pltpu.VMEM((C, FE), jnp.float32),
            pltpu.VMEM_SHARED((N, FE), jnp.float32),
            pltpu.VMEM_SHARED((N, FE), jnp.float32),
            pltpu.SemaphoreType.DMA((2,)),
            pltpu.SemaphoreType.DMA,
            pltpu.SemaphoreType.DMA,
        ],
    )
    def sc_scatter(attr_hbm, dest_hbm, zeros_hbm, ones_hbm,
                   psum_hbm, pcnt_hbm,
                   attr_v, idx_v, ones_v, sum_sh, cnt_sh,
                   lsem, ssem, csem):
        cid = lax.axis_index("c")
        sid = lax.axis_index("s")
        wid = sid * NC + cid
        base = sid * rows

        def start_loads(chunk_i, b):
            off = (wid * per_tile + chunk_i) * C
            pltpu.make_async_copy(
                dest_hbm.at[pl.ds(off, C)], idx_v.at[b], lsem.at[b]).start()
            pltpu.make_async_copy(
                attr_hbm.at[pl.ds(off, C)], attr_v.at[b], lsem.at[b]).start()

        def wait_loads(chunk_i, b):
            off = (wid * per_tile + chunk_i) * C
            pltpu.make_async_copy(
                dest_hbm.at[pl.ds(off, C)], idx_v.at[b], lsem.at[b]).wait()
            pltpu.make_async_copy(
                attr_hbm.at[pl.ds(off, C)], attr_v.at[b], lsem.at[b]).wait()

        pltpu.sync_copy(zeros_hbm.at[pl.ds(base, rows)],
                        sum_sh.at[pl.ds(base, rows)])
        pltpu.sync_copy(zeros_hbm.at[pl.ds(base, rows)],
                        cnt_sh.at[pl.ds(base, rows)])
        pltpu.sync_copy(ones_hbm, ones_v)
        plsc.subcore_barrier()
        start_loads(0, 0)

        def body(i, carry):
            b = lax.rem(i, 2)
            wait_loads(i, b)

            @pl.when(i + 1 < per_tile)
            def _():
                start_loads(i + 1, 1 - b)

            ds = pltpu.make_async_copy(attr_v.at[b], sum_sh.at[idx_v.at[b]],
                                       ssem)
            ds.start(add=True)
            dc = pltpu.make_async_copy(ones_v, cnt_sh.at[idx_v.at[b]], csem)
            dc.start(add=True)
            ds.wait()
            dc.wait()
            return carry

        lax.fori_loop(0, per_tile, body, None)
        plsc.subcore_barrier()
        pltpu.sync_copy(sum_sh.at[pl.ds(base, rows)],
                        psum_hbm.at[cid, pl.ds(base, rows)])
        pltpu.sync_copy(cnt_sh.at[pl.ds(base, rows)],
                        pcnt_hbm.at[cid, pl.ds(base, rows)])

    return sc_scatter


def _dense(x, a, psum, pcnt, u, W, b, batch3, blk=1000):
    N, FX = x.shape
    Bu, FU = u.shape
    FE = psum.shape[-1]
    FOUT = W.shape[1]
    nblk = N // blk

    def body(x_ref, a_ref, ps_ref, pc_ref, u_ref, w_ref, b_ref, bt_ref, o_ref):
        s = ps_ref[0] + ps_ref[1]
        cnt = pc_ref[0][:, 0:1] + pc_ref[1][:, 0:1]
        eagg = s / jnp.maximum(cnt, 1.0)
        bv = bt_ref[0, 0, :]
        oh = (bv[:, None] == lax.broadcasted_iota(jnp.int32, (blk, Bu), 1)
              ).astype(jnp.float32)
        ub = jnp.dot(oh, u_ref[...], preferred_element_type=jnp.float32)
        acc = jnp.dot(x_ref[...], w_ref[0:FX, :],
                      preferred_element_type=jnp.float32)
        acc += jnp.dot(a_ref[...], w_ref[FX:2 * FX, :],
                       preferred_element_type=jnp.float32)
        acc += jnp.dot(eagg, w_ref[2 * FX:2 * FX + FE, :],
                       preferred_element_type=jnp.float32)
        acc += jnp.dot(ub, w_ref[2 * FX + FE:, :],
                       preferred_element_type=jnp.float32)
        o_ref[...] = jnp.maximum(acc + b_ref[...][None, :], 0.0)

    return pl.pallas_call(
        body,
        grid=(nblk,),
        in_specs=[
            pl.BlockSpec((blk, FX), lambda i: (i, 0)),
            pl.BlockSpec((blk, FX), lambda i: (i, 0)),
            pl.BlockSpec((2, blk, FE), lambda i: (0, i, 0)),
            pl.BlockSpec((2, blk, FE), lambda i: (0, i, 0)),
            pl.BlockSpec((Bu, FU), lambda i: (0, 0)),
            pl.BlockSpec(W.shape, lambda i: (0, 0)),
            pl.BlockSpec(b.shape, lambda i: (0,)),
            pl.BlockSpec((1, 1, blk), lambda i: (i, 0, 0)),
        ],
        out_specs=pl.BlockSpec((blk, FOUT), lambda i: (i, 0)),
        out_shape=jax.ShapeDtypeStruct((N, FOUT), jnp.float32),
    )(x, a, psum, pcnt, u, W, b, batch3)


def kernel(x, a, edge_attr, u, W, b, edge_index, batch):
    E, FE = edge_attr.shape
    N = x.shape[0]
    C = 80
    info = plsc.get_sparse_core_info()
    align = info.num_subcores * 8
    n_pad = ((N + align - 1) // align) * align
    dest = edge_index[1]
    zeros = jnp.zeros((n_pad, FE), dtype=jnp.float32)
    ones = jnp.ones((C, FE), dtype=jnp.float32)
    psum, pcnt = _make_sc_scatter(E, n_pad, FE, C)(edge_attr, dest, zeros, ones)
    psum = psum[:, :N]
    pcnt = pcnt[:, :N]
    batch3 = batch.reshape(N // 1000, 1, 1000)
    return _dense(x, a, psum, pcnt, u, W, b, batch3, blk=1000)


# speedup vs baseline: 5.2373x; 1.0492x over previous
import functools

import jax
import jax.numpy as jnp
from jax import lax
from jax.experimental import pallas as pl
from jax.experimental.pallas import tpu as pltpu
from jax.experimental.pallas import tpu_sc as plsc


def _make_sc_scatter(E, N, FE, C=80, CW=8):
    info = plsc.get_sparse_core_info()
    NC, NS = info.num_cores, info.num_subcores
    NW = NC * NS
    n_chunks = E // C
    per_tile = n_chunks // NW
    rows = N // NS
    mesh = plsc.VectorSubcoreMesh(core_axis_name="c", subcore_axis_name="s")

    @functools.partial(
        pl.kernel,
        mesh=mesh,
        compiler_params=pltpu.CompilerParams(use_tc_tiling_on_sc=False),
        out_type=[
            jax.ShapeDtypeStruct((NC, N, FE), jnp.float32),
            jax.ShapeDtypeStruct((NC, N, CW), jnp.float32),
        ],
        scratch_types=[
            pltpu.VMEM((2, C, FE), jnp.float32),
            pltpu.VMEM((2, C), jnp.int32),
            pltpu.VMEM((C, CW), jnp.float32),
            pltpu.VMEM_SHARED((N, FE), jnp.float32),
            pltpu.VMEM_SHARED((N, CW), jnp.float32),
            pltpu.SemaphoreType.DMA((2,)),
            pltpu.SemaphoreType.DMA,
            pltpu.SemaphoreType.DMA,
        ],
    )
    def sc_scatter(attr_hbm, ei_hbm, zeros_hbm, zeros_c_hbm, ones_hbm,
                   psum_hbm, pcnt_hbm,
                   attr_v, idx_v, ones_v, sum_sh, cnt_sh,
                   lsem, ssem, csem):
        cid = lax.axis_index("c")
        sid = lax.axis_index("s")
        wid = sid * NC + cid
        base = sid * rows

        def start_loads(chunk_i, b):
            off = (wid * per_tile + chunk_i) * C
            pltpu.make_async_copy(
                ei_hbm.at[1, pl.ds(off, C)], idx_v.at[b], lsem.at[b]).start()
            pltpu.make_async_copy(
                attr_hbm.at[pl.ds(off, C)], attr_v.at[b], lsem.at[b]).start()

        def wait_loads(chunk_i, b):
            off = (wid * per_tile + chunk_i) * C
            pltpu.make_async_copy(
                ei_hbm.at[1, pl.ds(off, C)], idx_v.at[b], lsem.at[b]).wait()
            pltpu.make_async_copy(
                attr_hbm.at[pl.ds(off, C)], attr_v.at[b], lsem.at[b]).wait()

        pltpu.sync_copy(zeros_hbm.at[pl.ds(base, rows)],
                        sum_sh.at[pl.ds(base, rows)])
        pltpu.sync_copy(zeros_c_hbm.at[pl.ds(base, rows)],
                        cnt_sh.at[pl.ds(base, rows)])
        pltpu.sync_copy(ones_hbm, ones_v)
        plsc.subcore_barrier()
        start_loads(0, 0)

        def body(i, carry):
            b = lax.rem(i, 2)
            wait_loads(i, b)

            @pl.when(i + 1 < per_tile)
            def _():
                start_loads(i + 1, 1 - b)

            ds = pltpu.make_async_copy(attr_v.at[b], sum_sh.at[idx_v.at[b]],
                                       ssem)
            ds.start(add=True)
            dc = pltpu.make_async_copy(ones_v, cnt_sh.at[idx_v.at[b]], csem)
            dc.start(add=True)
            ds.wait()
            dc.wait()
            return carry

        lax.fori_loop(0, per_tile, body, None)
        plsc.subcore_barrier()
        pltpu.sync_copy(sum_sh.at[pl.ds(base, rows)],
                        psum_hbm.at[cid, pl.ds(base, rows)])
        pltpu.sync_copy(cnt_sh.at[pl.ds(base, rows)],
                        pcnt_hbm.at[cid, pl.ds(base, rows)])

    return sc_scatter


def _dense(x, a, psum, pcnt, u, W, b, batch3, blk=1000):
    N, FX = x.shape
    Bu, FU = u.shape
    FE = psum.shape[-1]
    CW = pcnt.shape[-1]
    FOUT = W.shape[1]
    nblk = N // blk

    def body(x_ref, a_ref, ps_ref, pc_ref, u_ref, w_ref, b_ref, bt_ref, o_ref):
        s = ps_ref[0] + ps_ref[1]
        cnt = pc_ref[0][:, 0:1] + pc_ref[1][:, 0:1]
        eagg = s / jnp.maximum(cnt, 1.0)
        bv = bt_ref[0, 0, :]
        oh = (bv[:, None] == lax.broadcasted_iota(jnp.int32, (blk, Bu), 1)
              ).astype(jnp.float32)
        ub = jnp.dot(oh, u_ref[...], preferred_element_type=jnp.float32)
        acc = jnp.dot(x_ref[...], w_ref[0:FX, :],
                      preferred_element_type=jnp.float32)
        acc += jnp.dot(a_ref[...], w_ref[FX:2 * FX, :],
                       preferred_element_type=jnp.float32)
        acc += jnp.dot(eagg, w_ref[2 * FX:2 * FX + FE, :],
                       preferred_element_type=jnp.float32)
        acc += jnp.dot(ub, w_ref[2 * FX + FE:, :],
                       preferred_element_type=jnp.float32)
        o_ref[...] = jnp.maximum(acc + b_ref[...][None, :], 0.0)

    return pl.pallas_call(
        body,
        grid=(nblk,),
        in_specs=[
            pl.BlockSpec((blk, FX), lambda i: (i, 0)),
            pl.BlockSpec((blk, FX), lambda i: (i, 0)),
            pl.BlockSpec((2, blk, FE), lambda i: (0, i, 0)),
            pl.BlockSpec((2, blk, CW), lambda i: (0, i, 0)),
            pl.BlockSpec((Bu, FU), lambda i: (0, 0)),
            pl.BlockSpec(W.shape, lambda i: (0, 0)),
            pl.BlockSpec(b.shape, lambda i: (0,)),
            pl.BlockSpec((1, 1, blk), lambda i: (i, 0, 0)),
        ],
        out_specs=pl.BlockSpec((blk, FOUT), lambda i: (i, 0)),
        out_shape=jax.ShapeDtypeStruct((N, FOUT), jnp.float32),
    )(x, a, psum, pcnt, u, W, b, batch3)


def kernel(x, a, edge_attr, u, W, b, edge_index, batch):
    E, FE = edge_attr.shape
    N = x.shape[0]
    C = 80
    CW = 8
    info = plsc.get_sparse_core_info()
    align = info.num_subcores * 8
    n_pad = ((N + align - 1) // align) * align
    zeros = jnp.zeros((n_pad, FE), dtype=jnp.float32)
    zeros_c = jnp.zeros((n_pad, CW), dtype=jnp.float32)
    ones = jnp.ones((C, CW), dtype=jnp.float32)
    psum, pcnt = _make_sc_scatter(E, n_pad, FE, C, CW)(
        edge_attr, edge_index, zeros, zeros_c, ones)
    batch3 = batch.reshape(N // 1000, 1, 1000)
    return _dense(x, a, psum, pcnt, u, W, b, batch3, blk=1000)


# speedup vs baseline: 5.2645x; 1.0052x over previous
import functools

import jax
import jax.numpy as jnp
from jax import lax
from jax.experimental import pallas as pl
from jax.experimental.pallas import tpu as pltpu
from jax.experimental.pallas import tpu_sc as plsc


def _make_sc_scatter(E, N, FE, C=80, CW=8):
    info = plsc.get_sparse_core_info()
    NC, NS = info.num_cores, info.num_subcores
    NW = NC * NS
    n_chunks = E // C
    per_tile = n_chunks // NW
    rows = N // NS
    mesh = plsc.VectorSubcoreMesh(core_axis_name="c", subcore_axis_name="s")

    @functools.partial(
        pl.kernel,
        mesh=mesh,
        compiler_params=pltpu.CompilerParams(use_tc_tiling_on_sc=False),
        out_type=[
            jax.ShapeDtypeStruct((NC, N, FE), jnp.float32),
            jax.ShapeDtypeStruct((NC, N, CW), jnp.float32),
        ],
        scratch_types=[
            pltpu.VMEM((2, C, FE), jnp.float32),
            pltpu.VMEM((2, C), jnp.int32),
            pltpu.VMEM((C, CW), jnp.float32),
            pltpu.VMEM_SHARED((N, FE), jnp.float32),
            pltpu.VMEM_SHARED((N, CW), jnp.float32),
            pltpu.SemaphoreType.DMA((2,)),
            pltpu.SemaphoreType.DMA,
            pltpu.SemaphoreType.DMA,
        ],
    )
    def sc_scatter(attr_hbm, ei_hbm, zeros_hbm, zeros_c_hbm, ones_hbm,
                   psum_hbm, pcnt_hbm,
                   attr_v, idx_v, ones_v, sum_sh, cnt_sh,
                   lsem, ssem, csem):
        cid = lax.axis_index("c")
        sid = lax.axis_index("s")
        wid = sid * NC + cid
        base = sid * rows

        def start_loads(chunk_i, b):
            off = (wid * per_tile + chunk_i) * C
            pltpu.make_async_copy(
                ei_hbm.at[1, pl.ds(off, C)], idx_v.at[b], lsem.at[b]).start()
            pltpu.make_async_copy(
                attr_hbm.at[pl.ds(off, C)], attr_v.at[b], lsem.at[b]).start()

        def wait_loads(chunk_i, b):
            off = (wid * per_tile + chunk_i) * C
            pltpu.make_async_copy(
                ei_hbm.at[1, pl.ds(off, C)], idx_v.at[b], lsem.at[b]).wait()
            pltpu.make_async_copy(
                attr_hbm.at[pl.ds(off, C)], attr_v.at[b], lsem.at[b]).wait()

        pltpu.sync_copy(zeros_hbm.at[pl.ds(base, rows)],
                        sum_sh.at[pl.ds(base, rows)])
        pltpu.sync_copy(zeros_c_hbm.at[pl.ds(base, rows)],
                        cnt_sh.at[pl.ds(base, rows)])
        pltpu.sync_copy(ones_hbm, ones_v)
        plsc.subcore_barrier()
        start_loads(0, 0)

        def body(i, carry):
            b = lax.rem(i, 2)
            wait_loads(i, b)

            @pl.when(i + 1 < per_tile)
            def _():
                start_loads(i + 1, 1 - b)

            ds = pltpu.make_async_copy(attr_v.at[b], sum_sh.at[idx_v.at[b]],
                                       ssem)
            ds.start(add=True)
            dc = pltpu.make_async_copy(ones_v, cnt_sh.at[idx_v.at[b]], csem)
            dc.start(add=True)
            ds.wait()
            dc.wait()
            return carry

        lax.fori_loop(0, per_tile, body, None)
        plsc.subcore_barrier()
        pltpu.sync_copy(sum_sh.at[pl.ds(base, rows)],
                        psum_hbm.at[cid, pl.ds(base, rows)])
        pltpu.sync_copy(cnt_sh.at[pl.ds(base, rows)],
                        pcnt_hbm.at[cid, pl.ds(base, rows)])

    return sc_scatter


def _dense_partial(x, a, u, W, b, batch3, blk=1000):
    N, FX = x.shape
    Bu, FU = u.shape
    FOUT = W.shape[1]
    FE = W.shape[0] - 2 * FX - FU
    nblk = N // blk

    def body(x_ref, a_ref, u_ref, w_ref, b_ref, bt_ref, o_ref):
        bv = bt_ref[0, 0, :]
        oh = (bv[:, None] == lax.broadcasted_iota(jnp.int32, (blk, Bu), 1)
              ).astype(jnp.float32)
        uw = jnp.dot(u_ref[...], w_ref[2 * FX + FE:, :],
                     preferred_element_type=jnp.float32)
        acc = jnp.dot(x_ref[...], w_ref[0:FX, :],
                      preferred_element_type=jnp.float32)
        acc += jnp.dot(a_ref[...], w_ref[FX:2 * FX, :],
                       preferred_element_type=jnp.float32)
        acc += jnp.dot(oh, uw, preferred_element_type=jnp.float32)
        o_ref[...] = acc + b_ref[...][None, :]

    return pl.pallas_call(
        body,
        grid=(nblk,),
        in_specs=[
            pl.BlockSpec((blk, FX), lambda i: (i, 0)),
            pl.BlockSpec((blk, FX), lambda i: (i, 0)),
            pl.BlockSpec((Bu, FU), lambda i: (0, 0)),
            pl.BlockSpec(W.shape, lambda i: (0, 0)),
            pl.BlockSpec(b.shape, lambda i: (0,)),
            pl.BlockSpec((1, 1, blk), lambda i: (i, 0, 0)),
        ],
        out_specs=pl.BlockSpec((blk, FOUT), lambda i: (i, 0)),
        out_shape=jax.ShapeDtypeStruct((N, FOUT), jnp.float32),
    )(x, a, u, W, b, batch3)


def _dense_final(partial, psum, pcnt, W, blk=1000):
    N, FOUT = partial.shape
    FE = psum.shape[-1]
    CW = pcnt.shape[-1]
    FX = (W.shape[0] - FE - 64) // 2
    nblk = N // blk

    def body(p_ref, ps_ref, pc_ref, w_ref, o_ref):
        s = ps_ref[0] + ps_ref[1]
        cnt = pc_ref[0][:, 0:1] + pc_ref[1][:, 0:1]
        eagg = s / jnp.maximum(cnt, 1.0)
        acc = p_ref[...] + jnp.dot(eagg, w_ref[2 * FX:2 * FX + FE, :],
                                   preferred_element_type=jnp.float32)
        o_ref[...] = jnp.maximum(acc, 0.0)

    return pl.pallas_call(
        body,
        grid=(nblk,),
        in_specs=[
            pl.BlockSpec((blk, FOUT), lambda i: (i, 0)),
            pl.BlockSpec((2, blk, FE), lambda i: (0, i, 0)),
            pl.BlockSpec((2, blk, CW), lambda i: (0, i, 0)),
            pl.BlockSpec(W.shape, lambda i: (0, 0)),
        ],
        out_specs=pl.BlockSpec((blk, FOUT), lambda i: (i, 0)),
        out_shape=jax.ShapeDtypeStruct((N, FOUT), jnp.float32),
    )(partial, psum, pcnt, W)


def kernel(x, a, edge_attr, u, W, b, edge_index, batch):
    E, FE = edge_attr.shape
    N = x.shape[0]
    C = 80
    CW = 8
    info = plsc.get_sparse_core_info()
    align = info.num_subcores * 8
    n_pad = ((N + align - 1) // align) * align
    zeros = jnp.zeros((n_pad, FE), dtype=jnp.float32)
    zeros_c = jnp.zeros((n_pad, CW), dtype=jnp.float32)
    ones = jnp.ones((C, CW), dtype=jnp.float32)
    psum, pcnt = _make_sc_scatter(E, n_pad, FE, C, CW)(
        edge_attr, edge_index, zeros, zeros_c, ones)
    batch3 = batch.reshape(N // 1000, 1, 1000)
    part = _dense_partial(x, a, u, W, b, batch3, blk=1000)
    return _dense_final(part, psum, pcnt, W, blk=1000)
